# baseline (speedup 1.0000x reference)
import functools

import jax
import jax.numpy as jnp
from jax.experimental import pallas as pl
from jax.experimental.pallas import tpu as pltpu


def _make_body(nE, nT):
    def _body(x_ref, wfe_ref, bfe_ref, wexp_ref, bexp_ref, out_ref,
              feat_ref, log_ref):
        t = pl.program_id(0)

        @pl.when(jnp.logical_and(t % nE == 0, t < nT))
        def _trunk():
            acc = jnp.dot(x_ref[...], wfe_ref[...],
                          preferred_element_type=jnp.float32)
            feat_ref[...] = jnp.maximum(acc + bfe_ref[...], 0.0
                                        ).astype(jnp.bfloat16)

        log_ref[t % 2] = jnp.dot(feat_ref[...], wexp_ref[0],
                                 preferred_element_type=jnp.float32
                                 ) + bexp_ref[0]
        l = log_ref[(t + 1) % 2]
        out_ref[0] = jnp.maximum(l, 0.0) + jnp.log1p(jnp.exp(-jnp.abs(l)))

    return _body


@functools.partial(jax.jit, static_argnames=("bn",))
def _geer(x, W_fe, b_fe, W_exp, b_exp, bn=1024):
    n, d = x.shape
    ne, _, c = W_exp.shape
    bn = min(bn, n)
    nn = n // bn
    nt = nn * ne
    xb = x.astype(jnp.bfloat16)
    wfeb = W_fe.astype(jnp.bfloat16)
    wexpb = W_exp.astype(jnp.bfloat16)
    bfe2 = b_fe.reshape(1, d).astype(jnp.float32)
    bexp2 = b_exp.reshape(ne, 1, c).astype(jnp.float32)

    def dot_i(t):
        return jnp.minimum(t, nt - 1) // ne

    def dot_e(t):
        return jnp.minimum(t, nt - 1) % ne

    def epi_t(t):
        return jnp.maximum(t - 1, 0)

    return pl.pallas_call(
        _make_body(ne, nt),
        grid=(nt + 1,),
        in_specs=[
            pl.BlockSpec((bn, d), lambda t: (dot_i(t), 0)),
            pl.BlockSpec((d, d), lambda t: (0, 0)),
            pl.BlockSpec((1, d), lambda t: (0, 0)),
            pl.BlockSpec((1, d, c), lambda t: (dot_e(t), 0, 0)),
            pl.BlockSpec((1, 1, c), lambda t: (dot_e(t), 0, 0)),
        ],
        out_specs=pl.BlockSpec(
            (1, bn, c), lambda t: (epi_t(t) % ne, epi_t(t) // ne, 0)),
        out_shape=jax.ShapeDtypeStruct((ne, n, c), jnp.float32),
        scratch_shapes=[
            pltpu.VMEM((bn, d), jnp.bfloat16),
            pltpu.VMEM((2, bn, c), jnp.float32),
        ],
        compiler_params=pltpu.CompilerParams(
            dimension_semantics=("arbitrary",),
        ),
    )(xb, wfeb, bfe2, wexpb, bexp2)


def kernel(x, W_fe, b_fe, W_exp, b_exp):
    return _geer(x, W_fe, b_fe, W_exp, b_exp)


# speedup vs baseline: 1.0017x; 1.0017x over previous
import functools

import jax
import jax.numpy as jnp
from jax.experimental import pallas as pl
from jax.experimental.pallas import tpu as pltpu


def _body(x_ref, wfe_ref, bfe_ref, wexp_ref, bexp_ref, out_ref, feat_ref):
    e = pl.program_id(1)

    @pl.when(e == 0)
    def _():
        acc = jnp.dot(x_ref[...], wfe_ref[...],
                      preferred_element_type=jnp.float32)
        acc = acc + bfe_ref[...]
        feat_ref[...] = jnp.maximum(acc, 0.0).astype(jnp.bfloat16)

    logits = jnp.dot(feat_ref[...], wexp_ref[0],
                     preferred_element_type=jnp.float32)
    logits = logits + bexp_ref[0]
    out_ref[0] = jnp.maximum(logits, 0.0) + jnp.log1p(jnp.exp(-jnp.abs(logits)))


@functools.partial(jax.jit, static_argnames=("bn",))
def _geer(x, W_fe, b_fe, W_exp, b_exp, bn=512):
    n, d = x.shape
    e, _, c = W_exp.shape
    bn = min(bn, n)
    xb = x.astype(jnp.bfloat16)
    wfeb = W_fe.astype(jnp.bfloat16)
    wexpb = W_exp.astype(jnp.bfloat16)
    bfe2 = b_fe.reshape(1, d).astype(jnp.float32)
    bexp2 = b_exp.reshape(e, 1, c).astype(jnp.float32)

    grid = (n // bn, e)
    return pl.pallas_call(
        _body,
        grid=grid,
        in_specs=[
            pl.BlockSpec((bn, d), lambda i, j: (i, 0)),
            pl.BlockSpec((d, d), lambda i, j: (0, 0)),
            pl.BlockSpec((1, d), lambda i, j: (0, 0)),
            pl.BlockSpec((1, d, c), lambda i, j: (j, 0, 0)),
            pl.BlockSpec((1, 1, c), lambda i, j: (j, 0, 0)),
        ],
        out_specs=pl.BlockSpec((1, bn, c), lambda i, j: (j, i, 0)),
        out_shape=jax.ShapeDtypeStruct((e, n, c), jnp.float32),
        scratch_shapes=[pltpu.VMEM((bn, d), jnp.bfloat16)],
        compiler_params=pltpu.CompilerParams(
            dimension_semantics=("arbitrary", "arbitrary"),
        ),
    )(xb, wfeb, bfe2, wexpb, bexp2)


def kernel(x, W_fe, b_fe, W_exp, b_exp):
    return _geer(x, W_fe, b_fe, W_exp, b_exp)


# speedup vs baseline: 1.0193x; 1.0175x over previous
import functools

import jax
import jax.numpy as jnp
from jax.experimental import pallas as pl
from jax.experimental.pallas import tpu as pltpu


def _body(x_ref, wfe_ref, bfe_ref, wexp_ref, bexp_ref, out_ref, feat_ref):
    e = pl.program_id(1)
    k = pl.program_id(2)

    @pl.when(jnp.logical_and(e == 0, k == 0))
    def _():
        acc = jnp.dot(x_ref[...], wfe_ref[...],
                      preferred_element_type=jnp.float32)
        acc = acc + bfe_ref[...]
        feat_ref[...] = jnp.maximum(acc, 0.0).astype(jnp.bfloat16)

    logits = jnp.dot(feat_ref[...], wexp_ref[0],
                     preferred_element_type=jnp.float32)
    logits = logits + bexp_ref[0]
    out_ref[0] = jnp.maximum(logits, 0.0) + jnp.log1p(jnp.exp(-jnp.abs(logits)))


@functools.partial(jax.jit, static_argnames=("bn", "bc"))
def _geer(x, W_fe, b_fe, W_exp, b_exp, bn=2048, bc=256):
    n, d = x.shape
    e, _, c = W_exp.shape
    bn = min(bn, n)
    bc = min(bc, c)
    xb = x.astype(jnp.bfloat16)
    wfeb = W_fe.astype(jnp.bfloat16)
    wexpb = W_exp.astype(jnp.bfloat16)
    bfe2 = b_fe.reshape(1, d).astype(jnp.float32)
    bexp2 = b_exp.reshape(e, 1, c).astype(jnp.float32)

    grid = (n // bn, e, c // bc)
    return pl.pallas_call(
        _body,
        grid=grid,
        in_specs=[
            pl.BlockSpec((bn, d), lambda i, j, k: (i, 0)),
            pl.BlockSpec((d, d), lambda i, j, k: (0, 0)),
            pl.BlockSpec((1, d), lambda i, j, k: (0, 0)),
            pl.BlockSpec((1, d, bc), lambda i, j, k: (j, 0, k)),
            pl.BlockSpec((1, 1, bc), lambda i, j, k: (j, 0, k)),
        ],
        out_specs=pl.BlockSpec((1, bn, bc), lambda i, j, k: (j, i, k)),
        out_shape=jax.ShapeDtypeStruct((e, n, c), jnp.float32),
        scratch_shapes=[pltpu.VMEM((bn, d), jnp.bfloat16)],
        compiler_params=pltpu.CompilerParams(
            dimension_semantics=("arbitrary", "arbitrary", "arbitrary"),
        ),
    )(xb, wfeb, bfe2, wexpb, bexp2)


def kernel(x, W_fe, b_fe, W_exp, b_exp):
    return _geer(x, W_fe, b_fe, W_exp, b_exp)
